# baseline (speedup 1.0000x reference)
import jax
import jax.numpy as jnp
from jax import lax
from jax.experimental import pallas as pl
from jax.experimental.pallas import tpu as pltpu
from jax.experimental.pallas import tpu_sc as plsc

N = 10000
D = 128
H = 16
C = 40
E = 320000
NC = 2
NS = 16
NW = NC * NS
CHUNK = 128
CPT = 80
NROWCH = 2608
E_PAD = NROWCH * CHUNK
ROWS_PT = N // NS


def _deg_body(dst_hbm, w_hbm, out_hbm, dst_v, w_v, stage_v, acc_sh):
    c = lax.axis_index("c")
    s = lax.axis_index("s")
    start = (c * NS + s) * CPT
    pltpu.sync_copy(dst_hbm.at[pl.ds(start, CPT)], dst_v)
    pltpu.sync_copy(w_hbm.at[pl.ds(start, CPT)], w_v)

    def zfill(i, carry):
        stage_v[pl.ds(i * 16, 16)] = jnp.zeros((16,), jnp.float32)
        return carry

    lax.fori_loop(0, 1024 // 16, zfill, 0)

    @pl.when(s < 10)
    def _():
        pltpu.sync_copy(stage_v.at[pl.ds(0, 1000)], acc_sh.at[pl.ds(s * 1000, 1000)])

    plsc.subcore_barrier()

    def chunk(j, carry):
        pltpu.sync_copy(w_v.at[j], acc_sh.at[dst_v.at[j]], add=True)
        return carry

    lax.fori_loop(0, CPT, chunk, 0)
    plsc.subcore_barrier()

    @pl.when(s < 10)
    def _():
        pltpu.sync_copy(acc_sh.at[pl.ds(s * 1000, 1000)], stage_v.at[pl.ds(0, 1000)])
        pltpu.sync_copy(stage_v.at[pl.ds(0, 1000)], out_hbm.at[pl.ds(c * N + s * 1000, 1000)])


NBUF = 4
M0 = 104
M1 = 56
CPT0 = M0


def _prop_body(s_hbm, src_hbm, dst_hbm, w_hbm, out_hbm,
               src_v, dst_v, w_v, rows0, rows1, rows2, rows3, sc0, sc1,
               stage_v, sem0, sem1, sem2, sem3, ssem0, ssem1, acc_sh):
    c = lax.axis_index("c")
    s = lax.axis_index("s")
    rows = (rows0, rows1, rows2, rows3)
    sems = (sem0, sem1, sem2, sem3)
    sc = (sc0, sc1)
    ssems = (ssem0, ssem1)
    m = jnp.where(c == 0, M0, M1)
    start = jnp.where(c == 0, s * M0, NS * M0 + s * M1)
    pltpu.sync_copy(src_hbm.at[pl.ds(start, CPT0)], src_v)
    pltpu.sync_copy(dst_hbm.at[pl.ds(start, CPT0)], dst_v)
    pltpu.sync_copy(w_hbm.at[pl.ds(start, CPT0)], w_v)

    def zfill(i, carry):
        stage_v[i, :] = jnp.zeros((16,), jnp.float32)
        return carry

    lax.fori_loop(0, 1000, zfill, 0)

    @pl.when(s < 10)
    def _():
        pltpu.sync_copy(stage_v, acc_sh.at[pl.ds(s * 1000, 1000)])

    plsc.subcore_barrier()

    def gather(j, b):
        pltpu.async_copy(s_hbm.at[src_v.at[j]], rows[b], sems[b])

    def drain(j, b):
        pltpu.make_async_copy(s_hbm.at[src_v.at[j]], rows[b], sems[b]).wait()

    def scale(j, b, p):
        def grp(g, c2):
            w16 = w_v[j, pl.ds(g * 16, 16)]
            base = g * 16
            for l in range(16):
                sc[p][base + l, :] = rows[b][base + l, :] * w16[l]
            return c2

        lax.fori_loop(0, CHUNK // 16, grp, 0)

    def scat(j, p):
        pltpu.async_copy(sc[p], acc_sh.at[dst_v.at[j]], ssems[p], add=True)

    def scat_wait(j, p):
        pltpu.make_async_copy(sc[p], acc_sh.at[dst_v.at[j]], ssems[p]).wait()

    def zs(i, carry):
        sc[0][i, :] = jnp.zeros((16,), jnp.float32)
        sc[1][i, :] = jnp.zeros((16,), jnp.float32)
        return carry

    lax.fori_loop(0, CHUNK, zs, 0)
    for p in range(2):
        scat(p, p)
    for b in range(NBUF):
        gather(b, b)

    def step(j, b, p):
        drain(j, b)
        scat_wait(j, p)
        scale(j, b, p)
        scat(j, p)

    def main_iter(i, carry):
        j = i * NBUF
        for b in range(NBUF):
            step(j + b, b, b % 2)
            gather(j + b + NBUF, b)
        return carry

    lax.fori_loop(0, m // NBUF - 1, main_iter, 0)
    for b in range(NBUF):
        j = m - NBUF + b
        step(j, b, b % 2)
    for p in range(2):
        scat_wait(m - 2 + p, p)

    plsc.subcore_barrier()

    @pl.when(s < 10)
    def _():
        pltpu.sync_copy(acc_sh.at[pl.ds(s * 1000, 1000)], stage_v)
        pltpu.sync_copy(stage_v, out_hbm.at[pl.ds(c * N + s * 1000, 1000)])


def _tc1_body(x_ref, w1_ref, degp_ref, dinv_ref, s1_ref):
    deg = 1.0 + degp_ref[0:N] + degp_ref[N:2 * N]
    dinv = lax.rsqrt(deg).reshape(N, 1)
    dinv_ref[...] = dinv
    t = jnp.dot(x_ref[...], w1_ref[...], preferred_element_type=jnp.float32)
    s1_ref[...] = t * dinv


def _tc2_body(up_ref, s1_ref, dinv_ref, s2_ref):
    u = up_ref[0:N, :] + up_ref[N:2 * N, :]
    dinv = dinv_ref[...]
    s2_ref[...] = (u + s1_ref[...]) * (dinv * dinv)


def _tc3_body(up_ref, s2_ref, dinv_ref, w2_ref, out_ref):
    u = up_ref[0:N, :] + up_ref[N:2 * N, :]
    g = (u + s2_ref[...]) * dinv_ref[...]
    z = jnp.dot(g, w2_ref[...], preferred_element_type=jnp.float32)
    m = jnp.max(z, axis=-1, keepdims=True)
    lse = m + jnp.log(jnp.sum(jnp.exp(z - m), axis=-1, keepdims=True))
    out_ref[...] = z - lse


def kernel(x, edge_index, edge_attr, W1, W2):
    f32 = jnp.float32
    src = edge_index[0].astype(jnp.int32)
    dst = edge_index[1].astype(jnp.int32)
    w = edge_attr.astype(f32)
    pad = E_PAD - E
    src_p = jnp.concatenate([src, jnp.zeros((pad,), jnp.int32)]).reshape(NROWCH, CHUNK)
    dst_p = jnp.concatenate([dst, jnp.zeros((pad,), jnp.int32)]).reshape(NROWCH, CHUNK)
    w_p = jnp.concatenate([w, jnp.zeros((pad,), f32)]).reshape(NROWCH, CHUNK)
    mesh = plsc.VectorSubcoreMesh(core_axis_name="c", subcore_axis_name="s",
                                  num_cores=NC, num_subcores=NS)

    sc_params = pltpu.CompilerParams(use_tc_tiling_on_sc=False)
    deg_call = pl.kernel(
        _deg_body,
        out_type=jax.ShapeDtypeStruct((2 * N,), f32),
        mesh=mesh,
        compiler_params=sc_params,
        scratch_types=[
            pltpu.VMEM((CPT, CHUNK), jnp.int32),
            pltpu.VMEM((CPT, CHUNK), f32),
            pltpu.VMEM((1024,), f32),
            pltpu.VMEM_SHARED((N,), f32),
        ],
    )
    prop_call = pl.kernel(
        _prop_body,
        out_type=jax.ShapeDtypeStruct((2 * N, H), f32),
        mesh=mesh,
        compiler_params=sc_params,
        scratch_types=[
            pltpu.VMEM((CPT0, CHUNK), jnp.int32),
            pltpu.VMEM((CPT0, CHUNK), jnp.int32),
            pltpu.VMEM((CPT0, CHUNK), f32),
            pltpu.VMEM((CHUNK, H), f32),
            pltpu.VMEM((CHUNK, H), f32),
            pltpu.VMEM((CHUNK, H), f32),
            pltpu.VMEM((CHUNK, H), f32),
            pltpu.VMEM((CHUNK, H), f32),
            pltpu.VMEM((CHUNK, H), f32),
            pltpu.VMEM((1000, H), f32),
            pltpu.SemaphoreType.DMA,
            pltpu.SemaphoreType.DMA,
            pltpu.SemaphoreType.DMA,
            pltpu.SemaphoreType.DMA,
            pltpu.SemaphoreType.DMA,
            pltpu.SemaphoreType.DMA,
            pltpu.VMEM_SHARED((N, H), f32),
        ],
    )

    tc1 = pl.pallas_call(
        _tc1_body,
        out_shape=[jax.ShapeDtypeStruct((N, 1), f32),
                   jax.ShapeDtypeStruct((N, H), f32)],
    )
    tc2 = pl.pallas_call(
        _tc2_body,
        out_shape=jax.ShapeDtypeStruct((N, H), f32),
    )
    tc3 = pl.pallas_call(
        _tc3_body,
        out_shape=jax.ShapeDtypeStruct((N, C), f32),
    )

    degp = deg_call(dst_p, w_p)
    dinv, s1 = tc1(x, W1, degp)
    u1p = prop_call(s1, src_p, dst_p, w_p)
    s2 = tc2(u1p, s1, dinv)
    u2p = prop_call(s2, src_p, dst_p, w_p)
    out = tc3(u2p, s2, dinv, W2)
    return out


# speedup vs baseline: 1.3596x; 1.3596x over previous
import jax
import jax.numpy as jnp
from jax import lax
from jax.experimental import pallas as pl
from jax.experimental.pallas import tpu as pltpu
from jax.experimental.pallas import tpu_sc as plsc

N = 10000
D = 128
H = 16
C = 40
E = 320000
NC = 2
NS = 16
NW = NC * NS
CHUNK = 128
CPT = 80
NROWCH = 2608
E_PAD = NROWCH * CHUNK
ROWS_PT = N // NS


def _deg_body(dst_hbm, w_hbm, out_hbm, dst_v, w_v, stage_v, acc_sh):
    c = lax.axis_index("c")
    s = lax.axis_index("s")
    start = (c * NS + s) * CPT
    pltpu.sync_copy(dst_hbm.at[pl.ds(start, CPT)], dst_v)
    pltpu.sync_copy(w_hbm.at[pl.ds(start, CPT)], w_v)

    def zfill(i, carry):
        stage_v[pl.ds(i * 16, 16)] = jnp.zeros((16,), jnp.float32)
        return carry

    lax.fori_loop(0, 1024 // 16, zfill, 0)

    @pl.when(s < 10)
    def _():
        pltpu.sync_copy(stage_v.at[pl.ds(0, 1000)], acc_sh.at[pl.ds(s * 1000, 1000)])

    plsc.subcore_barrier()

    def chunk(j, carry):
        pltpu.sync_copy(w_v.at[j], acc_sh.at[dst_v.at[j]], add=True)
        return carry

    lax.fori_loop(0, CPT, chunk, 0)
    plsc.subcore_barrier()

    @pl.when(s < 10)
    def _():
        pltpu.sync_copy(acc_sh.at[pl.ds(s * 1000, 1000)], stage_v.at[pl.ds(0, 1000)])
        pltpu.sync_copy(stage_v.at[pl.ds(0, 1000)], out_hbm.at[pl.ds(c * N + s * 1000, 1000)])


NBUF = 4
M0 = 80
M1 = 80
CPT0 = M0


def _prop_body(s_hbm, src_hbm, dst_hbm, w_hbm, out_hbm,
               src_v, dst_v, w_v, rows0, rows1, rows2, rows3, sc0, sc1,
               stage_v, sem0, sem1, sem2, sem3, ssem0, ssem1, acc_sh, s_sh):
    c = lax.axis_index("c")
    s = lax.axis_index("s")
    rows = (rows0, rows1, rows2, rows3)
    sems = (sem0, sem1, sem2, sem3)
    sc = (sc0, sc1)
    ssems = (ssem0, ssem1)
    m = jnp.where(c == 0, M0, M1)
    start = jnp.where(c == 0, s * M0, NS * M0 + s * M1)
    pltpu.sync_copy(src_hbm.at[pl.ds(start, CPT0)], src_v)
    pltpu.sync_copy(dst_hbm.at[pl.ds(start, CPT0)], dst_v)
    pltpu.sync_copy(w_hbm.at[pl.ds(start, CPT0)], w_v)

    @pl.when(s < 10)
    def _():
        pltpu.sync_copy(s_hbm.at[pl.ds(s * 1000, 1000)], stage_v)
        pltpu.sync_copy(stage_v, s_sh.at[pl.ds(s * 1000, 1000)])

    def zfill(i, carry):
        stage_v[i, :] = jnp.zeros((16,), jnp.float32)
        return carry

    lax.fori_loop(0, 1000, zfill, 0)

    @pl.when(s < 10)
    def _():
        pltpu.sync_copy(stage_v, acc_sh.at[pl.ds(s * 1000, 1000)])

    plsc.subcore_barrier()

    def gather(j, b):
        pltpu.async_copy(s_sh.at[src_v.at[j]], rows[b], sems[b])

    def drain(j, b):
        pltpu.make_async_copy(s_sh.at[src_v.at[j]], rows[b], sems[b]).wait()

    def scale(j, b, p):
        def grp(g, c2):
            w16 = w_v[j, pl.ds(g * 16, 16)]
            base = g * 16
            for l in range(16):
                sc[p][base + l, :] = rows[b][base + l, :] * w16[l]
            return c2

        lax.fori_loop(0, CHUNK // 16, grp, 0)

    def scat(j, p):
        pltpu.async_copy(sc[p], acc_sh.at[dst_v.at[j]], ssems[p], add=True)

    def scat_wait(j, p):
        pltpu.make_async_copy(sc[p], acc_sh.at[dst_v.at[j]], ssems[p]).wait()

    def zs(i, carry):
        sc[0][i, :] = jnp.zeros((16,), jnp.float32)
        sc[1][i, :] = jnp.zeros((16,), jnp.float32)
        return carry

    lax.fori_loop(0, CHUNK, zs, 0)
    for p in range(2):
        scat(p, p)
    for b in range(NBUF):
        gather(b, b)

    def step(j, b, p):
        drain(j, b)
        scat_wait(j, p)
        scale(j, b, p)
        scat(j, p)

    def main_iter(i, carry):
        j = i * NBUF
        for b in range(NBUF):
            step(j + b, b, b % 2)
            gather(j + b + NBUF, b)
        return carry

    lax.fori_loop(0, m // NBUF - 1, main_iter, 0)
    for b in range(NBUF):
        j = m - NBUF + b
        step(j, b, b % 2)
    for p in range(2):
        scat_wait(m - 2 + p, p)

    plsc.subcore_barrier()

    @pl.when(s < 10)
    def _():
        pltpu.sync_copy(acc_sh.at[pl.ds(s * 1000, 1000)], stage_v)
        pltpu.sync_copy(stage_v, out_hbm.at[pl.ds(c * N + s * 1000, 1000)])


def _tc1_body(x_ref, w1_ref, degp_ref, dinv_ref, s1_ref):
    deg = 1.0 + degp_ref[0:N] + degp_ref[N:2 * N]
    dinv = lax.rsqrt(deg).reshape(N, 1)
    dinv_ref[...] = dinv
    t = jnp.dot(x_ref[...], w1_ref[...], preferred_element_type=jnp.float32)
    s1_ref[...] = t * dinv


def _tc2_body(up_ref, s1_ref, dinv_ref, s2_ref):
    u = up_ref[0:N, :] + up_ref[N:2 * N, :]
    dinv = dinv_ref[...]
    s2_ref[...] = (u + s1_ref[...]) * (dinv * dinv)


def _tc3_body(up_ref, s2_ref, dinv_ref, w2_ref, out_ref):
    u = up_ref[0:N, :] + up_ref[N:2 * N, :]
    g = (u + s2_ref[...]) * dinv_ref[...]
    z = jnp.dot(g, w2_ref[...], preferred_element_type=jnp.float32)
    m = jnp.max(z, axis=-1, keepdims=True)
    lse = m + jnp.log(jnp.sum(jnp.exp(z - m), axis=-1, keepdims=True))
    out_ref[...] = z - lse


def kernel(x, edge_index, edge_attr, W1, W2):
    f32 = jnp.float32
    src = edge_index[0].astype(jnp.int32)
    dst = edge_index[1].astype(jnp.int32)
    w = edge_attr.astype(f32)
    pad = E_PAD - E
    src_p = jnp.concatenate([src, jnp.zeros((pad,), jnp.int32)]).reshape(NROWCH, CHUNK)
    dst_p = jnp.concatenate([dst, jnp.zeros((pad,), jnp.int32)]).reshape(NROWCH, CHUNK)
    w_p = jnp.concatenate([w, jnp.zeros((pad,), f32)]).reshape(NROWCH, CHUNK)
    mesh = plsc.VectorSubcoreMesh(core_axis_name="c", subcore_axis_name="s",
                                  num_cores=NC, num_subcores=NS)

    sc_params = pltpu.CompilerParams(use_tc_tiling_on_sc=False)
    deg_call = pl.kernel(
        _deg_body,
        out_type=jax.ShapeDtypeStruct((2 * N,), f32),
        mesh=mesh,
        compiler_params=sc_params,
        scratch_types=[
            pltpu.VMEM((CPT, CHUNK), jnp.int32),
            pltpu.VMEM((CPT, CHUNK), f32),
            pltpu.VMEM((1024,), f32),
            pltpu.VMEM_SHARED((N,), f32),
        ],
    )
    prop_call = pl.kernel(
        _prop_body,
        out_type=jax.ShapeDtypeStruct((2 * N, H), f32),
        mesh=mesh,
        compiler_params=sc_params,
        scratch_types=[
            pltpu.VMEM((CPT0, CHUNK), jnp.int32),
            pltpu.VMEM((CPT0, CHUNK), jnp.int32),
            pltpu.VMEM((CPT0, CHUNK), f32),
            pltpu.VMEM((CHUNK, H), f32),
            pltpu.VMEM((CHUNK, H), f32),
            pltpu.VMEM((CHUNK, H), f32),
            pltpu.VMEM((CHUNK, H), f32),
            pltpu.VMEM((CHUNK, H), f32),
            pltpu.VMEM((CHUNK, H), f32),
            pltpu.VMEM((1000, H), f32),
            pltpu.SemaphoreType.DMA,
            pltpu.SemaphoreType.DMA,
            pltpu.SemaphoreType.DMA,
            pltpu.SemaphoreType.DMA,
            pltpu.SemaphoreType.DMA,
            pltpu.SemaphoreType.DMA,
            pltpu.VMEM_SHARED((N, H), f32),
            pltpu.VMEM_SHARED((N, H), f32),
        ],
    )

    tc1 = pl.pallas_call(
        _tc1_body,
        out_shape=[jax.ShapeDtypeStruct((N, 1), f32),
                   jax.ShapeDtypeStruct((N, H), f32)],
    )
    tc2 = pl.pallas_call(
        _tc2_body,
        out_shape=jax.ShapeDtypeStruct((N, H), f32),
    )
    tc3 = pl.pallas_call(
        _tc3_body,
        out_shape=jax.ShapeDtypeStruct((N, C), f32),
    )

    degp = deg_call(dst_p, w_p)
    dinv, s1 = tc1(x, W1, degp)
    u1p = prop_call(s1, src_p, dst_p, w_p)
    s2 = tc2(u1p, s1, dinv)
    u2p = prop_call(s2, src_p, dst_p, w_p)
    out = tc3(u2p, s2, dinv, W2)
    return out


# speedup vs baseline: 1.5708x; 1.1553x over previous
import jax
import jax.numpy as jnp
from jax import lax
from jax.experimental import pallas as pl
from jax.experimental.pallas import tpu as pltpu
from jax.experimental.pallas import tpu_sc as plsc

N = 10000
D = 128
H = 16
C = 40
E = 320000
NC = 2
NS = 16
NW = NC * NS
CHUNK = 128
NROWCH = E // CHUNK
ROWS_PT = N // NS


def _edge_split(c, s):
    m = jnp.where(c == 0, M0, jnp.where(s == NS - 1, M0, M1))
    start = jnp.where(c == 0, s * M0, NS * M0 + s * M1)
    return m, start


def _deg_body(ei_hbm, w_hbm, out_hbm, dst_v, w_v, stage_v, acc_sh):
    c = lax.axis_index("c")
    s = lax.axis_index("s")
    m, start = _edge_split(c, s)
    pltpu.sync_copy(ei_hbm.at[1, pl.ds(start, CPT0)], dst_v)
    pltpu.sync_copy(w_hbm.at[pl.ds(start, CPT0)], w_v)

    def zfill(i, carry):
        stage_v[pl.ds(i * 16, 16)] = jnp.zeros((16,), jnp.float32)
        return carry

    lax.fori_loop(0, 1024 // 16, zfill, 0)

    @pl.when(s < 10)
    def _():
        pltpu.sync_copy(stage_v.at[pl.ds(0, 1000)], acc_sh.at[pl.ds(s * 1000, 1000)])

    plsc.subcore_barrier()

    def chunk(j, carry):
        pltpu.sync_copy(w_v.at[j], acc_sh.at[dst_v.at[j]], add=True)
        return carry

    lax.fori_loop(0, m, chunk, 0)
    plsc.subcore_barrier()

    @pl.when(s < 10)
    def _():
        pltpu.sync_copy(acc_sh.at[pl.ds(s * 1000, 1000)], stage_v.at[pl.ds(0, 1000)])
        pltpu.sync_copy(stage_v.at[pl.ds(0, 1000)], out_hbm.at[pl.ds(c * N + s * 1000, 1000)])


NBUF = 4
M0 = 80
M1 = 76
CPT0 = M0


def _prop_body(s_hbm, ei_hbm, w_hbm, out_hbm,
               src_v, dst_v, w_v, rows0, rows1, rows2, rows3, sc0, sc1,
               stage_v, sem0, sem1, sem2, sem3, ssem0, ssem1, acc_sh, s_sh):
    c = lax.axis_index("c")
    s = lax.axis_index("s")
    rows = (rows0, rows1, rows2, rows3)
    sems = (sem0, sem1, sem2, sem3)
    sc = (sc0, sc1)
    ssems = (ssem0, ssem1)
    m, start = _edge_split(c, s)
    pltpu.sync_copy(ei_hbm.at[0, pl.ds(start, CPT0)], src_v)
    pltpu.sync_copy(ei_hbm.at[1, pl.ds(start, CPT0)], dst_v)
    pltpu.sync_copy(w_hbm.at[pl.ds(start, CPT0)], w_v)

    @pl.when(s < 10)
    def _():
        pltpu.sync_copy(s_hbm.at[pl.ds(s * 1000, 1000)], stage_v)
        pltpu.sync_copy(stage_v, s_sh.at[pl.ds(s * 1000, 1000)])

    def zfill(i, carry):
        stage_v[i, :] = jnp.zeros((16,), jnp.float32)
        return carry

    lax.fori_loop(0, 1000, zfill, 0)

    @pl.when(s < 10)
    def _():
        pltpu.sync_copy(stage_v, acc_sh.at[pl.ds(s * 1000, 1000)])

    plsc.subcore_barrier()

    def gather(j, b):
        pltpu.async_copy(s_sh.at[src_v.at[j]], rows[b], sems[b])

    def drain(j, b):
        pltpu.make_async_copy(s_sh.at[src_v.at[j]], rows[b], sems[b]).wait()

    def scale(j, b, p):
        def grp(g, c2):
            w16 = w_v[j, pl.ds(g * 16, 16)]
            base = g * 16
            for l in range(16):
                sc[p][base + l, :] = rows[b][base + l, :] * w16[l]
            return c2

        lax.fori_loop(0, CHUNK // 16, grp, 0)

    def scat(j, p):
        pltpu.async_copy(sc[p], acc_sh.at[dst_v.at[j]], ssems[p], add=True)

    def scat_wait(j, p):
        pltpu.make_async_copy(sc[p], acc_sh.at[dst_v.at[j]], ssems[p]).wait()

    def zs(i, carry):
        sc[0][i, :] = jnp.zeros((16,), jnp.float32)
        sc[1][i, :] = jnp.zeros((16,), jnp.float32)
        return carry

    lax.fori_loop(0, CHUNK, zs, 0)
    for p in range(2):
        scat(p, p)
    for b in range(NBUF):
        gather(b, b)

    def step(j, b, p):
        drain(j, b)
        scat_wait(j, p)
        scale(j, b, p)
        scat(j, p)

    def main_iter(i, carry):
        j = i * NBUF
        for b in range(NBUF):
            step(j + b, b, b % 2)
            gather(j + b + NBUF, b)
        return carry

    lax.fori_loop(0, m // NBUF - 1, main_iter, 0)
    for b in range(NBUF):
        j = m - NBUF + b
        step(j, b, b % 2)
    for p in range(2):
        scat_wait(m - 2 + p, p)

    plsc.subcore_barrier()

    @pl.when(s < 10)
    def _():
        pltpu.sync_copy(acc_sh.at[pl.ds(s * 1000, 1000)], stage_v)
        pltpu.sync_copy(stage_v, out_hbm.at[pl.ds(c * N + s * 1000, 1000)])


def _tc1_body(x_ref, w1_ref, degp_ref, dinv_ref, s1_ref):
    deg = 1.0 + degp_ref[0:N] + degp_ref[N:2 * N]
    dinv = lax.rsqrt(deg).reshape(N, 1)
    dinv_ref[...] = dinv
    t = jnp.dot(x_ref[...], w1_ref[...], preferred_element_type=jnp.float32)
    s1_ref[...] = t * dinv


def _tc2_body(up_ref, s1_ref, dinv_ref, s2_ref):
    u = up_ref[0:N, :] + up_ref[N:2 * N, :]
    dinv = dinv_ref[...]
    s2_ref[...] = (u + s1_ref[...]) * (dinv * dinv)


def _tc3_body(up_ref, s2_ref, dinv_ref, w2_ref, out_ref):
    u = up_ref[0:N, :] + up_ref[N:2 * N, :]
    g = (u + s2_ref[...]) * dinv_ref[...]
    z = jnp.dot(g, w2_ref[...], preferred_element_type=jnp.float32)
    m = jnp.max(z, axis=-1, keepdims=True)
    lse = m + jnp.log(jnp.sum(jnp.exp(z - m), axis=-1, keepdims=True))
    out_ref[...] = z - lse


def kernel(x, edge_index, edge_attr, W1, W2):
    f32 = jnp.float32
    ei3 = edge_index.astype(jnp.int32).reshape(2, NROWCH, CHUNK)
    w_p = edge_attr.astype(f32).reshape(NROWCH, CHUNK)
    mesh = plsc.VectorSubcoreMesh(core_axis_name="c", subcore_axis_name="s",
                                  num_cores=NC, num_subcores=NS)

    sc_params = pltpu.CompilerParams(use_tc_tiling_on_sc=False)
    deg_call = pl.kernel(
        _deg_body,
        out_type=jax.ShapeDtypeStruct((2 * N,), f32),
        mesh=mesh,
        compiler_params=sc_params,
        scratch_types=[
            pltpu.VMEM((CPT0, CHUNK), jnp.int32),
            pltpu.VMEM((CPT0, CHUNK), f32),
            pltpu.VMEM((1024,), f32),
            pltpu.VMEM_SHARED((N,), f32),
        ],
    )
    prop_call = pl.kernel(
        _prop_body,
        out_type=jax.ShapeDtypeStruct((2 * N, H), f32),
        mesh=mesh,
        compiler_params=sc_params,
        scratch_types=[
            pltpu.VMEM((CPT0, CHUNK), jnp.int32),
            pltpu.VMEM((CPT0, CHUNK), jnp.int32),
            pltpu.VMEM((CPT0, CHUNK), f32),
            pltpu.VMEM((CHUNK, H), f32),
            pltpu.VMEM((CHUNK, H), f32),
            pltpu.VMEM((CHUNK, H), f32),
            pltpu.VMEM((CHUNK, H), f32),
            pltpu.VMEM((CHUNK, H), f32),
            pltpu.VMEM((CHUNK, H), f32),
            pltpu.VMEM((1000, H), f32),
            pltpu.SemaphoreType.DMA,
            pltpu.SemaphoreType.DMA,
            pltpu.SemaphoreType.DMA,
            pltpu.SemaphoreType.DMA,
            pltpu.SemaphoreType.DMA,
            pltpu.SemaphoreType.DMA,
            pltpu.VMEM_SHARED((N, H), f32),
            pltpu.VMEM_SHARED((N, H), f32),
        ],
    )

    tc1 = pl.pallas_call(
        _tc1_body,
        out_shape=[jax.ShapeDtypeStruct((N, 1), f32),
                   jax.ShapeDtypeStruct((N, H), f32)],
    )
    tc2 = pl.pallas_call(
        _tc2_body,
        out_shape=jax.ShapeDtypeStruct((N, H), f32),
    )
    tc3 = pl.pallas_call(
        _tc3_body,
        out_shape=jax.ShapeDtypeStruct((N, C), f32),
    )

    degp = deg_call(ei3, w_p)
    dinv, s1 = tc1(x, W1, degp)
    u1p = prop_call(s1, ei3, w_p)
    s2 = tc2(u1p, s1, dinv)
    u2p = prop_call(s2, ei3, w_p)
    out = tc3(u2p, s2, dinv, W2)
    return out


# speedup vs baseline: 1.5958x; 1.0159x over previous
import jax
import jax.numpy as jnp
from jax import lax
from jax.experimental import pallas as pl
from jax.experimental.pallas import tpu as pltpu
from jax.experimental.pallas import tpu_sc as plsc

N = 10000
D = 128
H = 16
C = 40
E = 320000
NC = 2
NS = 16
NW = NC * NS
CHUNK = 128
NROWCH = E // CHUNK
ROWS_PT = N // NS


def _edge_split(c, s):
    m = jnp.where(c == 0, M0, jnp.where(s == NS - 1, M0, M1))
    start = jnp.where(c == 0, s * M0, NS * M0 + s * M1)
    return m, start


def _deg_body(ei_hbm, w_hbm, out_hbm, dst_v, w_v, stage_v, dsem0, dsem1,
              acc_sh):
    c = lax.axis_index("c")
    s = lax.axis_index("s")
    dsems = (dsem0, dsem1)
    m, start = _edge_split(c, s)
    pltpu.sync_copy(ei_hbm.at[1, pl.ds(start, CPT0)], dst_v)
    pltpu.sync_copy(w_hbm.at[pl.ds(start, CPT0)], w_v)

    def zfill(i, carry):
        stage_v[pl.ds(i * 16, 16)] = jnp.zeros((16,), jnp.float32)
        return carry

    lax.fori_loop(0, 1024 // 16, zfill, 0)

    @pl.when(s < 10)
    def _():
        pltpu.sync_copy(stage_v.at[pl.ds(0, 1000)], acc_sh.at[pl.ds(s * 1000, 1000)])

    plsc.subcore_barrier()

    def scat(j, p):
        pltpu.async_copy(w_v.at[j], acc_sh.at[dst_v.at[j]], dsems[p], add=True)

    def scat_wait(j, p):
        pltpu.make_async_copy(w_v.at[j], acc_sh.at[dst_v.at[j]], dsems[p]).wait()

    for p in range(2):
        pltpu.async_copy(stage_v.at[pl.ds(0, CHUNK)],
                         acc_sh.at[dst_v.at[p]], dsems[p], add=True)

    def chunk2(i, carry):
        j = i * 2
        for p in range(2):
            scat_wait(j + p, p)
            scat(j + p, p)
        return carry

    lax.fori_loop(0, m // 2, chunk2, 0)
    for p in range(2):
        scat_wait(m - 2 + p, p)
    plsc.subcore_barrier()

    @pl.when(s < 10)
    def _():
        pltpu.sync_copy(acc_sh.at[pl.ds(s * 1000, 1000)], stage_v.at[pl.ds(0, 1000)])
        pltpu.sync_copy(stage_v.at[pl.ds(0, 1000)], out_hbm.at[pl.ds(c * N + s * 1000, 1000)])


_LANE_IDX = None
NBUF = 4
M0 = 80
M1 = 76
CPT0 = M0


def _prop_body(s_hbm, ei_hbm, w_hbm, out_hbm,
               src_v, dst_v, w_v, rows0, rows1, rows2, rows3, sc0, sc1,
               stage_v, sem0, sem1, sem2, sem3, ssem0, ssem1, acc_sh, s_sh):
    c = lax.axis_index("c")
    s = lax.axis_index("s")
    rows = (rows0, rows1, rows2, rows3)
    sems = (sem0, sem1, sem2, sem3)
    sc = (sc0, sc1)
    ssems = (ssem0, ssem1)
    global _LANE_IDX
    _LANE_IDX = [jnp.full((16, 1), l, jnp.int32) for l in range(16)]
    m, start = _edge_split(c, s)
    pltpu.sync_copy(ei_hbm.at[0, pl.ds(start, CPT0)], src_v)
    pltpu.sync_copy(ei_hbm.at[1, pl.ds(start, CPT0)], dst_v)
    pltpu.sync_copy(w_hbm.at[pl.ds(start, CPT0)], w_v)

    @pl.when(s < 10)
    def _():
        pltpu.sync_copy(s_hbm.at[pl.ds(s * 1000, 1000)], stage_v)
        pltpu.sync_copy(stage_v, s_sh.at[pl.ds(s * 1000, 1000)])

    def zfill(i, carry):
        stage_v[i, :] = jnp.zeros((16,), jnp.float32)
        return carry

    lax.fori_loop(0, 1000, zfill, 0)

    @pl.when(s < 10)
    def _():
        pltpu.sync_copy(stage_v, acc_sh.at[pl.ds(s * 1000, 1000)])

    plsc.subcore_barrier()

    def gather(j, b):
        pltpu.async_copy(s_sh.at[src_v.at[j]], rows[b], sems[b])

    def drain(j, b):
        pltpu.make_async_copy(s_sh.at[src_v.at[j]], rows[b], sems[b]).wait()

    gdn = lax.GatherDimensionNumbers(offset_dims=(), collapsed_slice_dims=(0,),
                                     start_index_map=(0,))

    def scale(j, b, p):
        def grp(g, c2):
            w16 = w_v[j, pl.ds(g * 16, 16)]
            base = g * 16
            for l in range(16):
                wl = lax.gather(w16, _LANE_IDX[l], gdn, (1,),
                                mode=lax.GatherScatterMode.PROMISE_IN_BOUNDS)
                sc[p][base + l, :] = rows[b][base + l, :] * wl
            return c2

        lax.fori_loop(0, CHUNK // 16, grp, 0)

    def scat(j, p):
        pltpu.async_copy(sc[p], acc_sh.at[dst_v.at[j]], ssems[p], add=True)

    def scat_wait(j, p):
        pltpu.make_async_copy(sc[p], acc_sh.at[dst_v.at[j]], ssems[p]).wait()

    def zs(i, carry):
        sc[0][i, :] = jnp.zeros((16,), jnp.float32)
        sc[1][i, :] = jnp.zeros((16,), jnp.float32)
        return carry

    lax.fori_loop(0, CHUNK, zs, 0)
    for p in range(2):
        scat(p, p)
    for b in range(NBUF):
        gather(b, b)

    def step(j, b, p):
        drain(j, b)
        scat_wait(j, p)
        scale(j, b, p)
        scat(j, p)

    def main_iter(i, carry):
        j = i * NBUF
        for b in range(NBUF):
            step(j + b, b, b % 2)
            gather(j + b + NBUF, b)
        return carry

    lax.fori_loop(0, m // NBUF - 1, main_iter, 0)
    for b in range(NBUF):
        j = m - NBUF + b
        step(j, b, b % 2)
    for p in range(2):
        scat_wait(m - 2 + p, p)

    plsc.subcore_barrier()

    @pl.when(s < 10)
    def _():
        pltpu.sync_copy(acc_sh.at[pl.ds(s * 1000, 1000)], stage_v)
        pltpu.sync_copy(stage_v, out_hbm.at[pl.ds(c * N + s * 1000, 1000)])


def _tc1_body(x_ref, w1_ref, degp_ref, dinv_ref, s1_ref):
    deg = 1.0 + degp_ref[0:N] + degp_ref[N:2 * N]
    dinv = lax.rsqrt(deg).reshape(N, 1)
    dinv_ref[...] = dinv
    t = jnp.dot(x_ref[...], w1_ref[...], preferred_element_type=jnp.float32)
    s1_ref[...] = t * dinv


def _tc2_body(up_ref, s1_ref, dinv_ref, s2_ref):
    u = up_ref[0:N, :] + up_ref[N:2 * N, :]
    dinv = dinv_ref[...]
    s2_ref[...] = (u + s1_ref[...]) * (dinv * dinv)


def _tc3_body(up_ref, s2_ref, dinv_ref, w2_ref, out_ref):
    u = up_ref[0:N, :] + up_ref[N:2 * N, :]
    g = (u + s2_ref[...]) * dinv_ref[...]
    z = jnp.dot(g, w2_ref[...], preferred_element_type=jnp.float32)
    m = jnp.max(z, axis=-1, keepdims=True)
    lse = m + jnp.log(jnp.sum(jnp.exp(z - m), axis=-1, keepdims=True))
    out_ref[...] = z - lse


def kernel(x, edge_index, edge_attr, W1, W2):
    f32 = jnp.float32
    ei3 = edge_index.astype(jnp.int32).reshape(2, NROWCH, CHUNK)
    w_p = edge_attr.astype(f32).reshape(NROWCH, CHUNK)
    mesh = plsc.VectorSubcoreMesh(core_axis_name="c", subcore_axis_name="s",
                                  num_cores=NC, num_subcores=NS)

    sc_params = pltpu.CompilerParams(use_tc_tiling_on_sc=False)
    deg_call = pl.kernel(
        _deg_body,
        out_type=jax.ShapeDtypeStruct((2 * N,), f32),
        mesh=mesh,
        compiler_params=sc_params,
        scratch_types=[
            pltpu.VMEM((CPT0, CHUNK), jnp.int32),
            pltpu.VMEM((CPT0, CHUNK), f32),
            pltpu.VMEM((1024,), f32),
            pltpu.SemaphoreType.DMA,
            pltpu.SemaphoreType.DMA,
            pltpu.VMEM_SHARED((N,), f32),
        ],
    )
    prop_call = pl.kernel(
        _prop_body,
        out_type=jax.ShapeDtypeStruct((2 * N, H), f32),
        mesh=mesh,
        compiler_params=sc_params,
        scratch_types=[
            pltpu.VMEM((CPT0, CHUNK), jnp.int32),
            pltpu.VMEM((CPT0, CHUNK), jnp.int32),
            pltpu.VMEM((CPT0, CHUNK), f32),
            pltpu.VMEM((CHUNK, H), f32),
            pltpu.VMEM((CHUNK, H), f32),
            pltpu.VMEM((CHUNK, H), f32),
            pltpu.VMEM((CHUNK, H), f32),
            pltpu.VMEM((CHUNK, H), f32),
            pltpu.VMEM((CHUNK, H), f32),
            pltpu.VMEM((1000, H), f32),
            pltpu.SemaphoreType.DMA,
            pltpu.SemaphoreType.DMA,
            pltpu.SemaphoreType.DMA,
            pltpu.SemaphoreType.DMA,
            pltpu.SemaphoreType.DMA,
            pltpu.SemaphoreType.DMA,
            pltpu.VMEM_SHARED((N, H), f32),
            pltpu.VMEM_SHARED((N, H), f32),
        ],
    )

    tc1 = pl.pallas_call(
        _tc1_body,
        out_shape=[jax.ShapeDtypeStruct((N, 1), f32),
                   jax.ShapeDtypeStruct((N, H), f32)],
    )
    tc2 = pl.pallas_call(
        _tc2_body,
        out_shape=jax.ShapeDtypeStruct((N, H), f32),
    )
    tc3 = pl.pallas_call(
        _tc3_body,
        out_shape=jax.ShapeDtypeStruct((N, C), f32),
    )

    degp = deg_call(ei3, w_p)
    dinv, s1 = tc1(x, W1, degp)
    u1p = prop_call(s1, ei3, w_p)
    s2 = tc2(u1p, s1, dinv)
    u2p = prop_call(s2, ei3, w_p)
    out = tc3(u2p, s2, dinv, W2)
    return out
